# baseline (speedup 1.0000x reference)
import jax
import jax.numpy as jnp
from jax.experimental import pallas as pl
from jax.experimental.pallas import tpu as pltpu

_B = 16384
_D = 128
_BLK = 2048
_GRID = _B // _BLK


def _fused_kernel(u_full_ref, d2_full_ref, d1_ref, d2_ref, v1_ref, v2_ref,
                  out_ref, m_ref):
    @pl.when(pl.program_id(0) == 0)
    def _():
        u = u_full_ref[...]
        m_ref[0] = jnp.max(u)
        m_ref[1] = jnp.max(u - d2_full_ref[...])

    m1 = m_ref[0]
    m2 = m_ref[1]
    out_ref[...] = (v2_ref[...] * jnp.minimum(d2_ref[...], m1)
                    + v1_ref[...] * jnp.minimum(d1_ref[...], m2))


def kernel(u, d1, d2, v1, v2):
    u_r = u.reshape(128, 128)
    d2_r = d2.reshape(128, 128)
    return pl.pallas_call(
        _fused_kernel,
        grid=(_GRID,),
        in_specs=[
            pl.BlockSpec((128, 128), lambda i: (0, 0)),
            pl.BlockSpec((128, 128), lambda i: (0, 0)),
            pl.BlockSpec((_BLK, 1), lambda i: (i, 0)),
            pl.BlockSpec((_BLK, 1), lambda i: (i, 0)),
            pl.BlockSpec((_BLK, _D), lambda i: (i, 0)),
            pl.BlockSpec((_BLK, _D), lambda i: (i, 0)),
        ],
        out_specs=pl.BlockSpec((_BLK, _D), lambda i: (i, 0)),
        out_shape=jax.ShapeDtypeStruct((_B, _D), jnp.float32),
        scratch_shapes=[pltpu.SMEM((2,), jnp.float32)],
    )(u_r, d2_r, d1, d2, v1, v2)
